# Initial kernel scaffold; baseline (speedup 1.0000x reference)
import jax
import jax.numpy as jnp
from jax import lax
from jax.experimental import pallas as pl
from jax.experimental.pallas import tpu as pltpu
from jax.experimental.pallas import tpu_sc as plsc

_B = 4
_S = 2048
_H = 768
_L = 16


def _last_token_gather(ids_hbm, table_hbm, out_hbm, ids_v, idx_v, rows_v, sem):
    wid = lax.axis_index("s") * 2 + lax.axis_index("c")

    @pl.when(wid == 0)
    def _():
        for b in range(_B):
            pltpu.sync_copy(ids_hbm.at[b, pl.ds(_S - _L, _L)], ids_v.at[b])
        rows = lax.rem(lax.iota(jnp.int32, _L), jnp.int32(_B))
        lanes = jnp.full((_L,), _L - 1, jnp.int32)
        idx_v[...] = plsc.load_gather(ids_v, [rows, lanes])
        pltpu.async_copy(table_hbm.at[idx_v], rows_v, sem).wait()
        pltpu.sync_copy(rows_v.at[pl.ds(0, _B)], out_hbm)


def kernel(input_ids, positions, emb_table):
    del positions
    ids32 = input_ids.astype(jnp.int32)
    mesh = plsc.VectorSubcoreMesh(core_axis_name="c", subcore_axis_name="s")
    return pl.kernel(
        _last_token_gather,
        mesh=mesh,
        out_type=jax.ShapeDtypeStruct((_B, _H), jnp.float32),
        scratch_types=[
            pltpu.VMEM((_B, _L), jnp.int32),
            pltpu.VMEM((_L,), jnp.int32),
            pltpu.VMEM((_L, _H), jnp.float32),
            pltpu.SemaphoreType.DMA,
        ],
    )(ids32, emb_table)


# speedup vs baseline: 1.8444x; 1.8444x over previous
import jax
import jax.numpy as jnp
from jax import lax
from jax.experimental import pallas as pl
from jax.experimental.pallas import tpu as pltpu
from jax.experimental.pallas import tpu_sc as plsc

_B = 4
_S = 2048
_H = 768
_L = 16


def _last_token_gather(ids_hbm, table_hbm, out_hbm, ids_v, idx_v, rows_v, sem):
    wid = lax.axis_index("s") * 2 + lax.axis_index("c")

    @pl.when(wid == 0)
    def _():
        for b in range(_B):
            pltpu.sync_copy(ids_hbm.at[b, pl.ds(_S - _L, _L)], ids_v.at[pl.ds(b * _L, _L)])
        lanes = lax.iota(jnp.int32, _L)
        acc = jnp.zeros((_L,), jnp.int32)
        for b in range(_B):
            chunk = ids_v[pl.ds(b * _L, _L)]
            acc = jnp.where(lax.rem(lanes, jnp.int32(_B)) == b, chunk[_L - 1], acc)
        idx_v[...] = acc
        pltpu.async_copy(table_hbm.at[idx_v], rows_v, sem).wait()
        pltpu.sync_copy(rows_v.at[pl.ds(0, _B)], out_hbm)


def kernel(input_ids, positions, emb_table):
    del positions
    ids32 = input_ids.astype(jnp.int32)
    mesh = plsc.VectorSubcoreMesh(core_axis_name="c", subcore_axis_name="s")
    return pl.kernel(
        _last_token_gather,
        mesh=mesh,
        out_type=jax.ShapeDtypeStruct((_B, _H), jnp.float32),
        scratch_types=[
            pltpu.VMEM((_B * _L,), jnp.int32),
            pltpu.VMEM((_L,), jnp.int32),
            pltpu.VMEM((_L, _H), jnp.float32),
            pltpu.SemaphoreType.DMA,
        ],
    )(ids32, emb_table)
